# baseline (speedup 1.0000x reference)
import functools

import jax
import jax.numpy as jnp
from jax import lax
from jax.experimental import pallas as pl
from jax.experimental.pallas import tpu as pltpu
from jax.experimental.pallas import tpu_sc as plsc

_F32 = jnp.float32
_BF16 = jnp.bfloat16

_SC_NC = 2
_SC_NS = 16
_SC_NW = _SC_NC * _SC_NS
_SC_LANES = 16
_IDX_CHUNK = 128


def _sc_gather_rows(tv, dv, t_tab, d_tab, t_hi, d_hi, dim):
    rows = tv.shape[0]
    per_w = rows // _SC_NW
    n_idx = per_w // _IDX_CHUNK
    n_vregs = _IDX_CHUNK // _SC_LANES

    mesh = plsc.VectorSubcoreMesh(core_axis_name="c", subcore_axis_name="s")

    @functools.partial(
        pl.kernel,
        mesh=mesh,
        out_type=[
            jax.ShapeDtypeStruct((rows, dim), _F32),
            jax.ShapeDtypeStruct((rows, dim), _F32),
        ],
        scratch_types=[
            pltpu.VMEM((per_w,), _F32),
            pltpu.VMEM((n_idx, _IDX_CHUNK), jnp.int32),
            pltpu.VMEM((per_w, dim), _F32),
            pltpu.SemaphoreType.DMA,
        ],
        compiler_params=pltpu.CompilerParams(
            needs_layout_passes=False, use_tc_tiling_on_sc=False),
    )
    def sc_kern(tv_hbm, dv_hbm, temb_hbm, demb_hbm, tout_hbm, dout_hbm,
                val_v, idx_v, rows_v, sem):
        wid = lax.axis_index("s") * _SC_NC + lax.axis_index("c")
        base = wid * per_w
        for src_hbm, tab_hbm, out_hbm, hi in (
            (tv_hbm, temb_hbm, tout_hbm, t_hi),
            (dv_hbm, demb_hbm, dout_hbm, d_hi),
        ):
            pltpu.sync_copy(src_hbm.at[pl.ds(base, per_w)], val_v)
            scale = jnp.float32(hi)
            for j in range(n_idx):
                for k in range(n_vregs):
                    v = val_v[pl.ds(j * _IDX_CHUNK + k * _SC_LANES,
                                    _SC_LANES)]
                    ix = jnp.clip((v * scale).astype(jnp.int32), 0, hi - 1)
                    idx_v[j, pl.ds(k * _SC_LANES, _SC_LANES)] = ix
            cps = [
                pltpu.async_copy(
                    tab_hbm.at[idx_v.at[j]],
                    rows_v.at[pl.ds(j * _IDX_CHUNK, _IDX_CHUNK)],
                    sem,
                )
                for j in range(n_idx)
            ]
            for cp in cps:
                cp.wait()
            pltpu.sync_copy(rows_v, out_hbm.at[pl.ds(base, per_w)])

    return sc_kern(tv, dv, t_tab.reshape(t_hi, dim), d_tab.reshape(d_hi, dim))


def kernel(history_data, future_data, batch_seen, epoch, train, node_emb, R,
           W_inv, t_emb, d_emb, W_ts, W_fc_in, betas, W1, b1, W2, b2, W_reg,
           b_reg):
    B, L, N, _ = history_data.shape
    D = node_emb.shape[1]
    OUT = W_reg.shape[0]
    TEMP = W1.shape[1]
    n_gnn = betas.shape[0]
    n_mlp = W1.shape[0]
    RW = N * B

    TN = 256
    RT = TN * B
    n_tiles = N // TN

    x_nbl = jnp.transpose(history_data[:, :, :, 0], (2, 0, 1)).reshape(RW, L)
    tv = jnp.transpose(history_data[:, -1, :, 1]).reshape(RW)
    dv = jnp.transpose(history_data[:, -1, :, 2]).reshape(RW)
    WtsT = jnp.transpose(W_ts)
    WfT = jnp.transpose(W_fc_in)
    W1T = jnp.transpose(W1, (0, 2, 1)).astype(_BF16)
    W2T = jnp.transpose(W2, (0, 2, 1)).astype(_BF16)
    b1r = b1[:, None, :]
    b2r = b2[:, None, :]
    WregT = jnp.transpose(W_reg).astype(_BF16)
    bregr = b_reg[None, :]

    t_rows, d_rows = _sc_gather_rows(tv, dv, t_emb.reshape(-1),
                                     d_emb.reshape(-1), t_emb.shape[0],
                                     d_emb.shape[0], D)

    def proj_body(ne_ref, r_ref, winv_ref, proj_ref):
        t = lax.dot_general(r_ref[...], ne_ref[...], (((1,), (0,)), ((), ())),
                            preferred_element_type=_F32)
        proj_ref[...] = lax.dot_general(
            winv_ref[...], t, (((1,), (0,)), ((), ())),
            preferred_element_type=_F32)

    proj = pl.pallas_call(
        proj_body,
        out_shape=jax.ShapeDtypeStruct((N, D), _F32),
    )(node_emb, R, W_inv)

    def adj_body(ptile_ref, pfull_ref, a_ref):
        s = lax.dot_general(ptile_ref[...], pfull_ref[...],
                            (((1,), (1,)), ((), ())),
                            preferred_element_type=_F32)
        s = jnp.maximum(s, 0.0)
        m = jnp.max(s, axis=1, keepdims=True)
        e = jnp.exp(s - m)
        a_ref[...] = (e / jnp.sum(e, axis=1, keepdims=True)).astype(_BF16)

    A = pl.pallas_call(
        adj_body,
        grid=(n_tiles,),
        in_specs=[
            pl.BlockSpec((TN, D), lambda i: (i, 0)),
            pl.BlockSpec((N, D), lambda i: (0, 0)),
        ],
        out_specs=pl.BlockSpec((TN, N), lambda i: (i, 0)),
        out_shape=jax.ShapeDtypeStruct((N, N), _BF16),
    )(proj, proj)

    def h0_body(x_ref, t_ref, d_ref, proj_ref, wts_ref, wf_ref, h0_ref,
                h0b_ref):
        tse = lax.dot_general(x_ref[...], wts_ref[...],
                              (((1,), (0,)), ((), ())),
                              preferred_element_type=_F32)
        node_e = jnp.broadcast_to(proj_ref[...][:, None, :],
                                  (TN, B, D)).reshape(RT, D)
        acc = lax.dot_general(tse, wf_ref[0:D], (((1,), (0,)), ((), ())),
                              preferred_element_type=_F32)
        acc += lax.dot_general(node_e, wf_ref[D:2 * D],
                               (((1,), (0,)), ((), ())),
                               preferred_element_type=_F32)
        acc += lax.dot_general(t_ref[...], wf_ref[2 * D:3 * D],
                               (((1,), (0,)), ((), ())),
                               preferred_element_type=_F32)
        acc += lax.dot_general(d_ref[...], wf_ref[3 * D:4 * D],
                               (((1,), (0,)), ((), ())),
                               preferred_element_type=_F32)
        h0_ref[...] = acc
        h0b_ref[...] = acc.astype(_BF16)

    h0_rows, h0b_rows = pl.pallas_call(
        h0_body,
        grid=(n_tiles,),
        in_specs=[
            pl.BlockSpec((RT, L), lambda i: (i, 0)),
            pl.BlockSpec((RT, D), lambda i: (i, 0)),
            pl.BlockSpec((RT, D), lambda i: (i, 0)),
            pl.BlockSpec((TN, D), lambda i: (i, 0)),
            pl.BlockSpec((L, D), lambda i: (0, 0)),
            pl.BlockSpec((4 * D, D), lambda i: (0, 0)),
        ],
        out_specs=[
            pl.BlockSpec((RT, D), lambda i: (i, 0)),
            pl.BlockSpec((RT, D), lambda i: (i, 0)),
        ],
        out_shape=[
            jax.ShapeDtypeStruct((RW, D), _F32),
            jax.ShapeDtypeStruct((RW, D), _BF16),
        ],
    )(x_nbl, t_rows, d_rows, proj, WtsT, WfT)

    BD = B * D
    h0f_mat = h0_rows.reshape(N, BD)
    h0b_mat = h0b_rows.reshape(N, BD)

    def prop_body(h0t_ref, hprev_ref, a_ref, betas_ref, out_ref, *, layer,
                  last):
        bta = betas_ref[layer]
        res = bta * h0t_ref[...].astype(_F32) + (1.0 - bta) * lax.dot_general(
            a_ref[...], hprev_ref[...], (((1,), (0,)), ((), ())),
            preferred_element_type=_F32)
        out_ref[...] = res if last else res.astype(_BF16)

    h_mat = h0b_mat
    for layer in range(n_gnn):
        last = layer == n_gnn - 1
        h_mat = pl.pallas_call(
            functools.partial(prop_body, layer=layer, last=last),
            grid=(n_tiles,),
            in_specs=[
                pl.BlockSpec((TN, BD), lambda i: (i, 0)),
                pl.BlockSpec((N, BD), lambda i: (0, 0)),
                pl.BlockSpec((TN, N), lambda i: (i, 0)),
                pl.BlockSpec(memory_space=pltpu.SMEM),
            ],
            out_specs=pl.BlockSpec((TN, BD), lambda i: (i, 0)),
            out_shape=jax.ShapeDtypeStruct((N, BD), _F32 if last else _BF16),
            compiler_params=pltpu.CompilerParams(
                vmem_limit_bytes=100 * 1024 * 1024),
        )(h0f_mat if last else h0b_mat, h_mat, A, betas)

    h2_rows = h_mat.reshape(RW, D)

    def mlp_body(x_ref, t_ref, d_ref, proj_ref, h2_ref, wts_ref, w1_ref,
                 b1_ref, w2_ref, b2_ref, wreg_ref, breg_ref, out_ref):
        tse = lax.dot_general(x_ref[...], wts_ref[...],
                              (((1,), (0,)), ((), ())),
                              preferred_element_type=_F32)
        node_e = jnp.broadcast_to(proj_ref[...][:, None, :],
                                  (TN, B, D)).reshape(RT, D)
        xx = jnp.concatenate(
            [tse, node_e, t_ref[...], d_ref[...], h2_ref[...]], axis=1)
        for l in range(n_mlp):
            hid = lax.dot_general(xx.astype(_BF16), w1_ref[l],
                                  (((1,), (0,)), ((), ())),
                                  preferred_element_type=_F32) + b1_ref[l]
            hid = jnp.maximum(hid, 0.0)
            xx = xx + lax.dot_general(hid.astype(_BF16), w2_ref[l],
                                      (((1,), (0,)), ((), ())),
                                      preferred_element_type=_F32) + b2_ref[l]
        out_ref[...] = lax.dot_general(xx.astype(_BF16), wreg_ref[...],
                                       (((1,), (0,)), ((), ())),
                                       preferred_element_type=_F32) + breg_ref[...]

    pred_rows = pl.pallas_call(
        mlp_body,
        grid=(n_tiles,),
        in_specs=[
            pl.BlockSpec((RT, L), lambda i: (i, 0)),
            pl.BlockSpec((RT, D), lambda i: (i, 0)),
            pl.BlockSpec((RT, D), lambda i: (i, 0)),
            pl.BlockSpec((TN, D), lambda i: (i, 0)),
            pl.BlockSpec((RT, D), lambda i: (i, 0)),
            pl.BlockSpec((L, D), lambda i: (0, 0)),
            pl.BlockSpec((n_mlp, TEMP, TEMP), lambda i: (0, 0, 0)),
            pl.BlockSpec((n_mlp, 1, TEMP), lambda i: (0, 0, 0)),
            pl.BlockSpec((n_mlp, TEMP, TEMP), lambda i: (0, 0, 0)),
            pl.BlockSpec((n_mlp, 1, TEMP), lambda i: (0, 0, 0)),
            pl.BlockSpec((TEMP, OUT), lambda i: (0, 0)),
            pl.BlockSpec((1, OUT), lambda i: (0, 0)),
        ],
        out_specs=pl.BlockSpec((RT, OUT), lambda i: (i, 0)),
        out_shape=jax.ShapeDtypeStruct((RW, OUT), _F32),
        compiler_params=pltpu.CompilerParams(
            vmem_limit_bytes=100 * 1024 * 1024),
    )(x_nbl, t_rows, d_rows, proj, h2_rows, WtsT, W1T, b1r, W2T, b2r, WregT,
      bregr)

    pred = jnp.transpose(pred_rows.reshape(N, B, OUT), (1, 2, 0))
    return pred[..., None]


# speedup vs baseline: 1.9276x; 1.9276x over previous
import functools

import jax
import jax.numpy as jnp
from jax import lax
from jax.experimental import pallas as pl
from jax.experimental.pallas import tpu as pltpu
from jax.experimental.pallas import tpu_sc as plsc

_F32 = jnp.float32
_BF16 = jnp.bfloat16

_SC_NC = 2
_SC_NS = 16
_SC_NW = _SC_NC * _SC_NS
_SC_LANES = 16
_IDX_CHUNK = 128


def _sc_gather_rows(tv, dv, t_tab, d_tab, t_hi, d_hi, dim):
    rows = tv.shape[0]
    per_w = rows // _SC_NW
    n_grp = per_w // _SC_LANES

    mesh = plsc.VectorSubcoreMesh(core_axis_name="c", subcore_axis_name="s")

    @functools.partial(
        pl.kernel,
        mesh=mesh,
        out_type=[
            jax.ShapeDtypeStruct((rows * dim,), _F32),
            jax.ShapeDtypeStruct((rows * dim,), _F32),
        ],
        scratch_types=[
            pltpu.VMEM(t_tab.shape, _F32),
            pltpu.VMEM(d_tab.shape, _F32),
            pltpu.VMEM((per_w,), _F32),
            pltpu.VMEM((per_w * dim,), _F32),
        ],
        compiler_params=pltpu.CompilerParams(needs_layout_passes=False),
    )
    def sc_kern(tv_hbm, dv_hbm, temb_hbm, demb_hbm, tout_hbm, dout_hbm,
                tab_t_v, tab_d_v, val_v, cmp_v):
        wid = lax.axis_index("s") * _SC_NC + lax.axis_index("c")
        base = wid * per_w
        lane_iota = lax.iota(jnp.int32, _SC_LANES)
        pos_base = lane_iota * dim
        rot_lo = [jnp.bitwise_and(lane_iota + c, dim - 1)
                  for c in range(dim // 2)]
        for src_hbm, tab_hbm, tab_v, out_hbm, hi in (
            (tv_hbm, temb_hbm, tab_t_v, tout_hbm, t_hi),
            (dv_hbm, demb_hbm, tab_d_v, dout_hbm, d_hi),
        ):
            pltpu.sync_copy(tab_hbm, tab_v)
            pltpu.sync_copy(src_hbm.at[pl.ds(base, per_w)], val_v)
            scale = jnp.float32(hi)

            @plsc.parallel_loop(0, n_grp, 1, unroll=8)
            def _(g):
                v = val_v[pl.ds(g * _SC_LANES, _SC_LANES)]
                idx16 = jnp.clip((v * scale).astype(jnp.int32), 0, hi - 1)
                fidx = idx16 * dim
                gbase = pos_base + g * (_SC_LANES * dim)
                for c in range(dim):
                    if c < dim // 2:
                        rot = rot_lo[c]
                    else:
                        rot = jnp.bitwise_xor(rot_lo[c - dim // 2],
                                              jnp.int32(dim // 2))
                    vals = plsc.load_gather(tab_v, [fidx + rot])
                    plsc.store_scatter(cmp_v, [gbase + rot], vals)
            pltpu.sync_copy(cmp_v, out_hbm.at[pl.ds(base * dim, per_w * dim)])

    return sc_kern(tv, dv, t_tab, d_tab)


def kernel(history_data, future_data, batch_seen, epoch, train, node_emb, R,
           W_inv, t_emb, d_emb, W_ts, W_fc_in, betas, W1, b1, W2, b2, W_reg,
           b_reg):
    B, L, N, _ = history_data.shape
    D = node_emb.shape[1]
    OUT = W_reg.shape[0]
    TEMP = W1.shape[1]
    n_gnn = betas.shape[0]
    n_mlp = W1.shape[0]
    RW = N * B

    TN = 256
    RT = TN * B
    n_tiles = N // TN

    x_nbl = jnp.transpose(history_data[:, :, :, 0], (2, 0, 1)).reshape(RW, L)
    tv = jnp.transpose(history_data[:, -1, :, 1]).reshape(RW)
    dv = jnp.transpose(history_data[:, -1, :, 2]).reshape(RW)
    WtsT = jnp.transpose(W_ts)
    WfT = jnp.transpose(W_fc_in)
    W1T = jnp.transpose(W1, (0, 2, 1)).astype(_BF16)
    W2T = jnp.transpose(W2, (0, 2, 1)).astype(_BF16)
    b1r = b1[:, None, :]
    b2r = b2[:, None, :]
    WregT = jnp.transpose(W_reg).astype(_BF16)
    bregr = b_reg[None, :]

    t_flat, d_flat = _sc_gather_rows(tv, dv, t_emb.reshape(-1),
                                     d_emb.reshape(-1), t_emb.shape[0],
                                     d_emb.shape[0], D)
    t_rows = t_flat.reshape(RW, D)
    d_rows = d_flat.reshape(RW, D)

    def proj_body(ne_ref, r_ref, winv_ref, proj_ref):
        t = lax.dot_general(r_ref[...], ne_ref[...], (((1,), (0,)), ((), ())),
                            preferred_element_type=_F32)
        proj_ref[...] = lax.dot_general(
            winv_ref[...], t, (((1,), (0,)), ((), ())),
            preferred_element_type=_F32)

    proj = pl.pallas_call(
        proj_body,
        out_shape=jax.ShapeDtypeStruct((N, D), _F32),
    )(node_emb, R, W_inv)

    def adj_body(ptile_ref, pfull_ref, a_ref):
        s = lax.dot_general(ptile_ref[...], pfull_ref[...],
                            (((1,), (1,)), ((), ())),
                            preferred_element_type=_F32)
        s = jnp.maximum(s, 0.0)
        m = jnp.max(s, axis=1, keepdims=True)
        e = jnp.exp(s - m)
        a_ref[...] = (e / jnp.sum(e, axis=1, keepdims=True)).astype(_BF16)

    A = pl.pallas_call(
        adj_body,
        grid=(n_tiles,),
        in_specs=[
            pl.BlockSpec((TN, D), lambda i: (i, 0)),
            pl.BlockSpec((N, D), lambda i: (0, 0)),
        ],
        out_specs=pl.BlockSpec((TN, N), lambda i: (i, 0)),
        out_shape=jax.ShapeDtypeStruct((N, N), _BF16),
    )(proj, proj)

    def h0_body(x_ref, t_ref, d_ref, proj_ref, wts_ref, wf_ref, h0_ref,
                h0b_ref):
        tse = lax.dot_general(x_ref[...], wts_ref[...],
                              (((1,), (0,)), ((), ())),
                              preferred_element_type=_F32)
        node_e = jnp.broadcast_to(proj_ref[...][:, None, :],
                                  (TN, B, D)).reshape(RT, D)
        acc = lax.dot_general(tse, wf_ref[0:D], (((1,), (0,)), ((), ())),
                              preferred_element_type=_F32)
        acc += lax.dot_general(node_e, wf_ref[D:2 * D],
                               (((1,), (0,)), ((), ())),
                               preferred_element_type=_F32)
        acc += lax.dot_general(t_ref[...], wf_ref[2 * D:3 * D],
                               (((1,), (0,)), ((), ())),
                               preferred_element_type=_F32)
        acc += lax.dot_general(d_ref[...], wf_ref[3 * D:4 * D],
                               (((1,), (0,)), ((), ())),
                               preferred_element_type=_F32)
        h0_ref[...] = acc
        h0b_ref[...] = acc.astype(_BF16)

    h0_rows, h0b_rows = pl.pallas_call(
        h0_body,
        grid=(n_tiles,),
        in_specs=[
            pl.BlockSpec((RT, L), lambda i: (i, 0)),
            pl.BlockSpec((RT, D), lambda i: (i, 0)),
            pl.BlockSpec((RT, D), lambda i: (i, 0)),
            pl.BlockSpec((TN, D), lambda i: (i, 0)),
            pl.BlockSpec((L, D), lambda i: (0, 0)),
            pl.BlockSpec((4 * D, D), lambda i: (0, 0)),
        ],
        out_specs=[
            pl.BlockSpec((RT, D), lambda i: (i, 0)),
            pl.BlockSpec((RT, D), lambda i: (i, 0)),
        ],
        out_shape=[
            jax.ShapeDtypeStruct((RW, D), _F32),
            jax.ShapeDtypeStruct((RW, D), _BF16),
        ],
    )(x_nbl, t_rows, d_rows, proj, WtsT, WfT)

    BD = B * D
    h0f_mat = h0_rows.reshape(N, BD)
    h0b_mat = h0b_rows.reshape(N, BD)

    def prop_body(h0b_tile_ref, h0f_tile_ref, h0b_full_ref, a_ref, betas_ref,
                  out_ref, h1_scr):
        i = pl.program_id(0)
        ph = i // n_tiles
        j = lax.rem(i, n_tiles)

        @pl.when(ph == 0)
        def _():
            acc = lax.dot_general(a_ref[...], h0b_full_ref[...],
                                  (((1,), (0,)), ((), ())),
                                  preferred_element_type=_F32)
            b0 = betas_ref[0]
            h1 = b0 * h0b_tile_ref[...].astype(_F32) + (1.0 - b0) * acc
            h1_scr[pl.ds(j * TN, TN), :] = h1.astype(_BF16)
            out_ref[...] = h1

        @pl.when(ph == 1)
        def _():
            acc = lax.dot_general(a_ref[...], h1_scr[...],
                                  (((1,), (0,)), ((), ())),
                                  preferred_element_type=_F32)
            b1_ = betas_ref[n_gnn - 1]
            out_ref[...] = b1_ * h0f_tile_ref[...] + (1.0 - b1_) * acc

    h_mat = pl.pallas_call(
        prop_body,
        grid=(n_gnn * n_tiles,),
        in_specs=[
            pl.BlockSpec((TN, BD), lambda i: (i % n_tiles, 0)),
            pl.BlockSpec((TN, BD), lambda i: (i % n_tiles, 0)),
            pl.BlockSpec((N, BD), lambda i: (0, 0)),
            pl.BlockSpec((TN, N), lambda i: (i % n_tiles, 0)),
            pl.BlockSpec(memory_space=pltpu.SMEM),
        ],
        out_specs=pl.BlockSpec((TN, BD), lambda i: (i % n_tiles, 0)),
        out_shape=jax.ShapeDtypeStruct((N, BD), _F32),
        scratch_shapes=[pltpu.VMEM((N, BD), _BF16)],
        compiler_params=pltpu.CompilerParams(
            vmem_limit_bytes=100 * 1024 * 1024),
    )(h0b_mat, h0f_mat, h0b_mat, A, betas)

    h2_rows = h_mat.reshape(RW, D)

    def mlp_body(x_ref, t_ref, d_ref, proj_ref, h2_ref, wts_ref, w1_ref,
                 b1_ref, w2_ref, b2_ref, wreg_ref, breg_ref, out_ref):
        tse = lax.dot_general(x_ref[...], wts_ref[...],
                              (((1,), (0,)), ((), ())),
                              preferred_element_type=_F32)
        node_e = jnp.broadcast_to(proj_ref[...][:, None, :],
                                  (TN, B, D)).reshape(RT, D)
        xx = jnp.concatenate(
            [tse, node_e, t_ref[...], d_ref[...], h2_ref[...]], axis=1)
        for l in range(n_mlp):
            hid = lax.dot_general(xx.astype(_BF16), w1_ref[l],
                                  (((1,), (0,)), ((), ())),
                                  preferred_element_type=_F32) + b1_ref[l]
            hid = jnp.maximum(hid, 0.0)
            xx = xx + lax.dot_general(hid.astype(_BF16), w2_ref[l],
                                      (((1,), (0,)), ((), ())),
                                      preferred_element_type=_F32) + b2_ref[l]
        out_ref[...] = lax.dot_general(xx.astype(_BF16), wreg_ref[...],
                                       (((1,), (0,)), ((), ())),
                                       preferred_element_type=_F32) + breg_ref[...]

    pred_rows = pl.pallas_call(
        mlp_body,
        grid=(n_tiles,),
        in_specs=[
            pl.BlockSpec((RT, L), lambda i: (i, 0)),
            pl.BlockSpec((RT, D), lambda i: (i, 0)),
            pl.BlockSpec((RT, D), lambda i: (i, 0)),
            pl.BlockSpec((TN, D), lambda i: (i, 0)),
            pl.BlockSpec((RT, D), lambda i: (i, 0)),
            pl.BlockSpec((L, D), lambda i: (0, 0)),
            pl.BlockSpec((n_mlp, TEMP, TEMP), lambda i: (0, 0, 0)),
            pl.BlockSpec((n_mlp, 1, TEMP), lambda i: (0, 0, 0)),
            pl.BlockSpec((n_mlp, TEMP, TEMP), lambda i: (0, 0, 0)),
            pl.BlockSpec((n_mlp, 1, TEMP), lambda i: (0, 0, 0)),
            pl.BlockSpec((TEMP, OUT), lambda i: (0, 0)),
            pl.BlockSpec((1, OUT), lambda i: (0, 0)),
        ],
        out_specs=pl.BlockSpec((RT, OUT), lambda i: (i, 0)),
        out_shape=jax.ShapeDtypeStruct((RW, OUT), _F32),
        compiler_params=pltpu.CompilerParams(
            vmem_limit_bytes=100 * 1024 * 1024),
    )(x_nbl, t_rows, d_rows, proj, h2_rows, WtsT, W1T, b1r, W2T, b2r, WregT,
      bregr)

    pred = jnp.transpose(pred_rows.reshape(N, B, OUT), (1, 2, 0))
    return pred[..., None]


# speedup vs baseline: 1.9504x; 1.0119x over previous
import functools

import jax
import jax.numpy as jnp
from jax import lax
from jax.experimental import pallas as pl
from jax.experimental.pallas import tpu as pltpu
from jax.experimental.pallas import tpu_sc as plsc

_F32 = jnp.float32
_BF16 = jnp.bfloat16

_SC_NC = 2
_SC_NS = 16
_SC_NW = _SC_NC * _SC_NS
_SC_LANES = 16
_IDX_CHUNK = 128


def _sc_gather_rows(tv, dv, t_tab, d_tab, t_hi, d_hi, dim):
    rows = tv.shape[0]
    per_w = rows // _SC_NW
    n_grp = per_w // _SC_LANES

    mesh = plsc.VectorSubcoreMesh(core_axis_name="c", subcore_axis_name="s")

    @functools.partial(
        pl.kernel,
        mesh=mesh,
        out_type=[
            jax.ShapeDtypeStruct((rows * dim,), _F32),
            jax.ShapeDtypeStruct((rows * dim,), _F32),
        ],
        scratch_types=[
            pltpu.VMEM(t_tab.shape, _F32),
            pltpu.VMEM(d_tab.shape, _F32),
            pltpu.VMEM((per_w,), _F32),
            pltpu.VMEM((per_w * dim,), _F32),
        ],
        compiler_params=pltpu.CompilerParams(needs_layout_passes=False),
    )
    def sc_kern(tv_hbm, dv_hbm, temb_hbm, demb_hbm, tout_hbm, dout_hbm,
                tab_t_v, tab_d_v, val_v, cmp_v):
        wid = lax.axis_index("s") * _SC_NC + lax.axis_index("c")
        base = wid * per_w
        lane_iota = lax.iota(jnp.int32, _SC_LANES)
        pos_base = lane_iota * dim
        rot_lo = [jnp.bitwise_and(lane_iota + c, dim - 1)
                  for c in range(dim // 2)]
        for src_hbm, tab_hbm, tab_v, out_hbm, hi in (
            (tv_hbm, temb_hbm, tab_t_v, tout_hbm, t_hi),
            (dv_hbm, demb_hbm, tab_d_v, dout_hbm, d_hi),
        ):
            pltpu.sync_copy(tab_hbm, tab_v)
            pltpu.sync_copy(src_hbm.at[pl.ds(base, per_w)], val_v)
            scale = jnp.float32(hi)

            @plsc.parallel_loop(0, n_grp, 1, unroll=8)
            def _(g):
                v = val_v[pl.ds(g * _SC_LANES, _SC_LANES)]
                idx16 = jnp.clip((v * scale).astype(jnp.int32), 0, hi - 1)
                fidx = idx16 * dim
                gbase = pos_base + g * (_SC_LANES * dim)
                for c in range(dim):
                    if c < dim // 2:
                        rot = rot_lo[c]
                    else:
                        rot = jnp.bitwise_xor(rot_lo[c - dim // 2],
                                              jnp.int32(dim // 2))
                    vals = plsc.load_gather(tab_v, [fidx + rot])
                    plsc.store_scatter(cmp_v, [gbase + rot], vals)
            pltpu.sync_copy(cmp_v, out_hbm.at[pl.ds(base * dim, per_w * dim)])

    return sc_kern(tv, dv, t_tab, d_tab)


def kernel(history_data, future_data, batch_seen, epoch, train, node_emb, R,
           W_inv, t_emb, d_emb, W_ts, W_fc_in, betas, W1, b1, W2, b2, W_reg,
           b_reg):
    B, L, N, _ = history_data.shape
    D = node_emb.shape[1]
    OUT = W_reg.shape[0]
    TEMP = W1.shape[1]
    n_gnn = betas.shape[0]
    n_mlp = W1.shape[0]
    RW = N * B

    TN = 256
    RT = TN * B
    n_tiles = N // TN

    x_nbl = jnp.transpose(history_data[:, :, :, 0], (2, 0, 1)).reshape(RW, L)
    tv = jnp.transpose(history_data[:, -1, :, 1]).reshape(RW)
    dv = jnp.transpose(history_data[:, -1, :, 2]).reshape(RW)
    WtsT = jnp.transpose(W_ts)
    WfT = jnp.transpose(W_fc_in)
    W1T = jnp.transpose(W1, (0, 2, 1)).astype(_BF16)
    W2T = jnp.transpose(W2, (0, 2, 1)).astype(_BF16)
    b1r = b1[:, None, :]
    b2r = b2[:, None, :]
    WregT = jnp.transpose(W_reg).astype(_BF16)
    bregr = b_reg[None, :]

    t_flat, d_flat = _sc_gather_rows(tv, dv, t_emb.reshape(-1),
                                     d_emb.reshape(-1), t_emb.shape[0],
                                     d_emb.shape[0], D)
    t_rows = t_flat.reshape(RW, D)
    d_rows = d_flat.reshape(RW, D)

    def proj_body(ne_ref, r_ref, winv_ref, proj_ref):
        t = lax.dot_general(r_ref[...], ne_ref[...], (((1,), (0,)), ((), ())),
                            preferred_element_type=_F32)
        proj_ref[...] = lax.dot_general(
            winv_ref[...], t, (((1,), (0,)), ((), ())),
            preferred_element_type=_F32)

    proj = pl.pallas_call(
        proj_body,
        out_shape=jax.ShapeDtypeStruct((N, D), _F32),
    )(node_emb, R, W_inv)

    def h0_body(x_ref, t_ref, d_ref, proj_ref, wts_ref, wf_ref, h0_ref,
                h0b_ref):
        tse = lax.dot_general(x_ref[...], wts_ref[...],
                              (((1,), (0,)), ((), ())),
                              preferred_element_type=_F32)
        node_e = jnp.broadcast_to(proj_ref[...][:, None, :],
                                  (TN, B, D)).reshape(RT, D)
        acc = lax.dot_general(tse, wf_ref[0:D], (((1,), (0,)), ((), ())),
                              preferred_element_type=_F32)
        acc += lax.dot_general(node_e, wf_ref[D:2 * D],
                               (((1,), (0,)), ((), ())),
                               preferred_element_type=_F32)
        acc += lax.dot_general(t_ref[...], wf_ref[2 * D:3 * D],
                               (((1,), (0,)), ((), ())),
                               preferred_element_type=_F32)
        acc += lax.dot_general(d_ref[...], wf_ref[3 * D:4 * D],
                               (((1,), (0,)), ((), ())),
                               preferred_element_type=_F32)
        h0_ref[...] = acc
        h0b_ref[...] = acc.astype(_BF16)

    h0_rows, h0b_rows = pl.pallas_call(
        h0_body,
        grid=(n_tiles,),
        in_specs=[
            pl.BlockSpec((RT, L), lambda i: (i, 0)),
            pl.BlockSpec((RT, D), lambda i: (i, 0)),
            pl.BlockSpec((RT, D), lambda i: (i, 0)),
            pl.BlockSpec((TN, D), lambda i: (i, 0)),
            pl.BlockSpec((L, D), lambda i: (0, 0)),
            pl.BlockSpec((4 * D, D), lambda i: (0, 0)),
        ],
        out_specs=[
            pl.BlockSpec((RT, D), lambda i: (i, 0)),
            pl.BlockSpec((RT, D), lambda i: (i, 0)),
        ],
        out_shape=[
            jax.ShapeDtypeStruct((RW, D), _F32),
            jax.ShapeDtypeStruct((RW, D), _BF16),
        ],
    )(x_nbl, t_rows, d_rows, proj, WtsT, WfT)

    BD = B * D
    h0f_mat = h0_rows.reshape(N, BD)
    h0b_mat = h0b_rows.reshape(N, BD)

    def adjprop_body(ptile_ref, pfull_ref, h0f_tile_ref, h0b_full_ref,
                     betas_ref, out_ref, a_scr, h1_scr):
        i = pl.program_id(0)
        ph = i // n_tiles
        j = lax.rem(i, n_tiles)
        rows = pl.ds(j * TN, TN)

        @pl.when(ph == 0)
        def _():
            s = lax.dot_general(ptile_ref[...], pfull_ref[...],
                                (((1,), (1,)), ((), ())),
                                preferred_element_type=_F32)
            s = jnp.maximum(s, 0.0)
            m = jnp.max(s, axis=1, keepdims=True)
            e = jnp.exp(s - m)
            a_scr[rows, :] = (e / jnp.sum(e, axis=1, keepdims=True)
                              ).astype(_BF16)
            out_ref[...] = h0f_tile_ref[...]

        @pl.when(ph == 1)
        def _():
            acc = lax.dot_general(a_scr[rows, :], h0b_full_ref[...],
                                  (((1,), (0,)), ((), ())),
                                  preferred_element_type=_F32)
            b0 = betas_ref[0]
            h1 = (b0 * h0b_full_ref[rows, :].astype(_F32)
                  + (1.0 - b0) * acc)
            h1_scr[rows, :] = h1.astype(_BF16)
            out_ref[...] = h1

        @pl.when(ph == 2)
        def _():
            acc = lax.dot_general(a_scr[rows, :], h1_scr[...],
                                  (((1,), (0,)), ((), ())),
                                  preferred_element_type=_F32)
            b1_ = betas_ref[n_gnn - 1]
            out_ref[...] = b1_ * h0f_tile_ref[...] + (1.0 - b1_) * acc

    h_mat = pl.pallas_call(
        adjprop_body,
        grid=((n_gnn + 1) * n_tiles,),
        in_specs=[
            pl.BlockSpec((TN, D), lambda i: (i % n_tiles, 0)),
            pl.BlockSpec((N, D), lambda i: (0, 0)),
            pl.BlockSpec((TN, BD), lambda i: (i % n_tiles, 0)),
            pl.BlockSpec((N, BD), lambda i: (0, 0)),
            pl.BlockSpec(memory_space=pltpu.SMEM),
        ],
        out_specs=pl.BlockSpec((TN, BD), lambda i: (i % n_tiles, 0)),
        out_shape=jax.ShapeDtypeStruct((N, BD), _F32),
        scratch_shapes=[
            pltpu.VMEM((N, N), _BF16),
            pltpu.VMEM((N, BD), _BF16),
        ],
        compiler_params=pltpu.CompilerParams(
            vmem_limit_bytes=110 * 1024 * 1024),
    )(proj, proj, h0f_mat, h0b_mat, betas)

    h2_rows = h_mat.reshape(RW, D)

    def mlp_body(x_ref, t_ref, d_ref, proj_ref, h2_ref, wts_ref, w1_ref,
                 b1_ref, w2_ref, b2_ref, wreg_ref, breg_ref, out_ref):
        tse = lax.dot_general(x_ref[...], wts_ref[...],
                              (((1,), (0,)), ((), ())),
                              preferred_element_type=_F32)
        node_e = jnp.broadcast_to(proj_ref[...][:, None, :],
                                  (TN, B, D)).reshape(RT, D)
        xx = jnp.concatenate(
            [tse, node_e, t_ref[...], d_ref[...], h2_ref[...]], axis=1)
        for l in range(n_mlp):
            hid = lax.dot_general(xx.astype(_BF16), w1_ref[l],
                                  (((1,), (0,)), ((), ())),
                                  preferred_element_type=_F32) + b1_ref[l]
            hid = jnp.maximum(hid, 0.0)
            xx = xx + lax.dot_general(hid.astype(_BF16), w2_ref[l],
                                      (((1,), (0,)), ((), ())),
                                      preferred_element_type=_F32) + b2_ref[l]
        out_ref[...] = lax.dot_general(xx.astype(_BF16), wreg_ref[...],
                                       (((1,), (0,)), ((), ())),
                                       preferred_element_type=_F32) + breg_ref[...]

    pred_rows = pl.pallas_call(
        mlp_body,
        grid=(n_tiles,),
        in_specs=[
            pl.BlockSpec((RT, L), lambda i: (i, 0)),
            pl.BlockSpec((RT, D), lambda i: (i, 0)),
            pl.BlockSpec((RT, D), lambda i: (i, 0)),
            pl.BlockSpec((TN, D), lambda i: (i, 0)),
            pl.BlockSpec((RT, D), lambda i: (i, 0)),
            pl.BlockSpec((L, D), lambda i: (0, 0)),
            pl.BlockSpec((n_mlp, TEMP, TEMP), lambda i: (0, 0, 0)),
            pl.BlockSpec((n_mlp, 1, TEMP), lambda i: (0, 0, 0)),
            pl.BlockSpec((n_mlp, TEMP, TEMP), lambda i: (0, 0, 0)),
            pl.BlockSpec((n_mlp, 1, TEMP), lambda i: (0, 0, 0)),
            pl.BlockSpec((TEMP, OUT), lambda i: (0, 0)),
            pl.BlockSpec((1, OUT), lambda i: (0, 0)),
        ],
        out_specs=pl.BlockSpec((RT, OUT), lambda i: (i, 0)),
        out_shape=jax.ShapeDtypeStruct((RW, OUT), _F32),
        compiler_params=pltpu.CompilerParams(
            vmem_limit_bytes=100 * 1024 * 1024),
    )(x_nbl, t_rows, d_rows, proj, h2_rows, WtsT, W1T, b1r, W2T, b2r, WregT,
      bregr)

    pred = jnp.transpose(pred_rows.reshape(N, B, OUT), (1, 2, 0))
    return pred[..., None]


# speedup vs baseline: 2.1128x; 1.0832x over previous
import functools

import jax
import jax.numpy as jnp
from jax import lax
from jax.experimental import pallas as pl
from jax.experimental.pallas import tpu as pltpu
from jax.experimental.pallas import tpu_sc as plsc

_F32 = jnp.float32
_BF16 = jnp.bfloat16

_SC_NC = 2
_SC_NS = 16
_SC_NW = _SC_NC * _SC_NS
_SC_LANES = 16
_IDX_CHUNK = 128


def _sc_gather_rows(tv, dv, t_tab, d_tab, t_hi, d_hi, dim):
    rows = tv.shape[0]
    per_w = rows // _SC_NW
    n_grp = per_w // _SC_LANES

    mesh = plsc.VectorSubcoreMesh(core_axis_name="c", subcore_axis_name="s")

    @functools.partial(
        pl.kernel,
        mesh=mesh,
        out_type=[
            jax.ShapeDtypeStruct((rows * dim,), _F32),
            jax.ShapeDtypeStruct((rows * dim,), _F32),
        ],
        scratch_types=[
            pltpu.VMEM(t_tab.shape, _F32),
            pltpu.VMEM(d_tab.shape, _F32),
            pltpu.VMEM((per_w,), _F32),
            pltpu.VMEM((per_w * dim,), _F32),
        ],
        compiler_params=pltpu.CompilerParams(needs_layout_passes=False),
    )
    def sc_kern(tv_hbm, dv_hbm, temb_hbm, demb_hbm, tout_hbm, dout_hbm,
                tab_t_v, tab_d_v, val_v, cmp_v):
        wid = lax.axis_index("s") * _SC_NC + lax.axis_index("c")
        base = wid * per_w
        lane_iota = lax.iota(jnp.int32, _SC_LANES)
        pos_base = lane_iota * dim
        rot_lo = [jnp.bitwise_and(lane_iota + c, dim - 1)
                  for c in range(dim // 2)]
        for src_hbm, tab_hbm, tab_v, out_hbm, hi in (
            (tv_hbm, temb_hbm, tab_t_v, tout_hbm, t_hi),
            (dv_hbm, demb_hbm, tab_d_v, dout_hbm, d_hi),
        ):
            pltpu.sync_copy(tab_hbm, tab_v)
            pltpu.sync_copy(src_hbm.at[pl.ds(base, per_w)], val_v)
            scale = jnp.float32(hi)

            @plsc.parallel_loop(0, n_grp, 1, unroll=8)
            def _(g):
                v = val_v[pl.ds(g * _SC_LANES, _SC_LANES)]
                idx16 = jnp.clip((v * scale).astype(jnp.int32), 0, hi - 1)
                fidx = idx16 * dim
                gbase = pos_base + g * (_SC_LANES * dim)
                for c in range(dim):
                    if c < dim // 2:
                        rot = rot_lo[c]
                    else:
                        rot = jnp.bitwise_xor(rot_lo[c - dim // 2],
                                              jnp.int32(dim // 2))
                    vals = plsc.load_gather(tab_v, [fidx + rot])
                    plsc.store_scatter(cmp_v, [gbase + rot], vals)
            pltpu.sync_copy(cmp_v, out_hbm.at[pl.ds(base * dim, per_w * dim)])

    return sc_kern(tv, dv, t_tab, d_tab)


def kernel(history_data, future_data, batch_seen, epoch, train, node_emb, R,
           W_inv, t_emb, d_emb, W_ts, W_fc_in, betas, W1, b1, W2, b2, W_reg,
           b_reg):
    B, L, N, _ = history_data.shape
    D = node_emb.shape[1]
    OUT = W_reg.shape[0]
    TEMP = W1.shape[1]
    n_gnn = betas.shape[0]
    n_mlp = W1.shape[0]
    RW = N * B

    TN = 256
    RT = TN * B
    n_tiles = N // TN

    x_bln = history_data[:, :, :, 0]
    tv = jnp.transpose(history_data[:, -1, :, 1]).reshape(RW)
    dv = jnp.transpose(history_data[:, -1, :, 2]).reshape(RW)
    WtsT = jnp.transpose(W_ts)
    WfT = jnp.transpose(W_fc_in)
    W1T = jnp.transpose(W1, (0, 2, 1)).astype(_BF16)
    W2T = jnp.transpose(W2, (0, 2, 1)).astype(_BF16)
    b1r = b1[:, None, :]
    b2r = b2[:, None, :]
    WregT = jnp.transpose(W_reg).astype(_BF16)
    bregr = b_reg[None, :]

    t_flat, d_flat = _sc_gather_rows(tv, dv, t_emb.reshape(-1),
                                     d_emb.reshape(-1), t_emb.shape[0],
                                     d_emb.shape[0], D)
    t_rows = t_flat.reshape(RW, D)
    d_rows = d_flat.reshape(RW, D)

    def proj_body(ne_ref, r_ref, winv_ref, proj_ref):
        t = lax.dot_general(r_ref[...], ne_ref[...], (((1,), (0,)), ((), ())),
                            preferred_element_type=_F32)
        proj_ref[...] = lax.dot_general(
            winv_ref[...], t, (((1,), (0,)), ((), ())),
            preferred_element_type=_F32)

    proj = pl.pallas_call(
        proj_body,
        out_shape=jax.ShapeDtypeStruct((N, D), _F32),
    )(node_emb, R, W_inv)

    def tse_from_raw(x_ref, wts_ref, scr):
        for b in range(B):
            scr[:, b, :] = lax.dot_general(x_ref[b], wts_ref[...],
                                           (((0,), (0,)), ((), ())),
                                           preferred_element_type=_F32)
        return scr[...].reshape(RT, D)

    def h0_body(x_ref, t_ref, d_ref, proj_ref, wts_ref, wf_ref, h0b_ref,
                tse_scr):
        tse = tse_from_raw(x_ref, wts_ref, tse_scr)
        node_e = jnp.broadcast_to(proj_ref[...][:, None, :],
                                  (TN, B, D)).reshape(RT, D)
        acc = lax.dot_general(tse, wf_ref[0:D], (((1,), (0,)), ((), ())),
                              preferred_element_type=_F32)
        acc += lax.dot_general(node_e, wf_ref[D:2 * D],
                               (((1,), (0,)), ((), ())),
                               preferred_element_type=_F32)
        acc += lax.dot_general(t_ref[...], wf_ref[2 * D:3 * D],
                               (((1,), (0,)), ((), ())),
                               preferred_element_type=_F32)
        acc += lax.dot_general(d_ref[...], wf_ref[3 * D:4 * D],
                               (((1,), (0,)), ((), ())),
                               preferred_element_type=_F32)
        h0b_ref[...] = acc.astype(_BF16)

    h0b_rows = pl.pallas_call(
        h0_body,
        grid=(n_tiles,),
        in_specs=[
            pl.BlockSpec((B, L, TN), lambda i: (0, 0, i)),
            pl.BlockSpec((RT, D), lambda i: (i, 0)),
            pl.BlockSpec((RT, D), lambda i: (i, 0)),
            pl.BlockSpec((TN, D), lambda i: (i, 0)),
            pl.BlockSpec((L, D), lambda i: (0, 0)),
            pl.BlockSpec((4 * D, D), lambda i: (0, 0)),
        ],
        out_specs=pl.BlockSpec((RT, D), lambda i: (i, 0)),
        out_shape=jax.ShapeDtypeStruct((RW, D), _BF16),
        scratch_shapes=[pltpu.VMEM((TN, B, D), _F32)],
    )(x_bln, t_rows, d_rows, proj, WtsT, WfT)

    BD = B * D
    h0b_mat = h0b_rows.reshape(N, BD)

    def adjprop_body(ptile_ref, pfull_ref, h0b_full_ref, betas_ref, out_ref,
                     a_scr, h1_scr):
        i = pl.program_id(0)
        ph = i // n_tiles
        j = lax.rem(i, n_tiles)
        rows = pl.ds(j * TN, TN)

        @pl.when(ph == 0)
        def _():
            s = lax.dot_general(ptile_ref[...], pfull_ref[...],
                                (((1,), (1,)), ((), ())),
                                preferred_element_type=_F32)
            s = jnp.maximum(s, 0.0)
            m = jnp.max(s, axis=1, keepdims=True)
            e = jnp.exp(s - m)
            a_scr[rows, :] = (e / jnp.sum(e, axis=1, keepdims=True)
                              ).astype(_BF16)
            out_ref[...] = h0b_full_ref[rows, :]

        @pl.when(ph == 1)
        def _():
            acc = lax.dot_general(a_scr[rows, :], h0b_full_ref[...],
                                  (((1,), (0,)), ((), ())),
                                  preferred_element_type=_F32)
            b0 = betas_ref[0]
            h1 = (b0 * h0b_full_ref[rows, :].astype(_F32)
                  + (1.0 - b0) * acc)
            h1_scr[rows, :] = h1.astype(_BF16)
            out_ref[...] = h1.astype(_BF16)

        @pl.when(ph == 2)
        def _():
            acc = lax.dot_general(a_scr[rows, :], h1_scr[...],
                                  (((1,), (0,)), ((), ())),
                                  preferred_element_type=_F32)
            b1_ = betas_ref[n_gnn - 1]
            h2 = (b1_ * h0b_full_ref[rows, :].astype(_F32)
                  + (1.0 - b1_) * acc)
            out_ref[...] = h2.astype(_BF16)

    h_mat = pl.pallas_call(
        adjprop_body,
        grid=((n_gnn + 1) * n_tiles,),
        in_specs=[
            pl.BlockSpec((TN, D), lambda i: (i % n_tiles, 0)),
            pl.BlockSpec((N, D), lambda i: (0, 0)),
            pl.BlockSpec((N, BD), lambda i: (0, 0)),
            pl.BlockSpec(memory_space=pltpu.SMEM),
        ],
        out_specs=pl.BlockSpec((TN, BD), lambda i: (i % n_tiles, 0)),
        out_shape=jax.ShapeDtypeStruct((N, BD), _BF16),
        scratch_shapes=[
            pltpu.VMEM((N, N), _BF16),
            pltpu.VMEM((N, BD), _BF16),
        ],
        compiler_params=pltpu.CompilerParams(
            vmem_limit_bytes=110 * 1024 * 1024),
    )(proj, proj, h0b_mat, betas)

    h2_rows = h_mat.reshape(RW, D)

    def mlp_body(x_ref, t_ref, d_ref, proj_ref, h2_ref, wts_ref, w1_ref,
                 b1_ref, w2_ref, b2_ref, wreg_ref, breg_ref, out_ref,
                 tse_scr):
        tse = tse_from_raw(x_ref, wts_ref, tse_scr)
        node_e = jnp.broadcast_to(proj_ref[...][:, None, :],
                                  (TN, B, D)).reshape(RT, D)
        xx = jnp.concatenate(
            [tse, node_e, t_ref[...], d_ref[...],
             h2_ref[...].astype(_F32)], axis=1)
        for l in range(n_mlp):
            hid = lax.dot_general(xx.astype(_BF16), w1_ref[l],
                                  (((1,), (0,)), ((), ())),
                                  preferred_element_type=_F32) + b1_ref[l]
            hid = jnp.maximum(hid, 0.0)
            xx = xx + lax.dot_general(hid.astype(_BF16), w2_ref[l],
                                      (((1,), (0,)), ((), ())),
                                      preferred_element_type=_F32) + b2_ref[l]
        out_ref[...] = lax.dot_general(xx.astype(_BF16), wreg_ref[...],
                                       (((1,), (0,)), ((), ())),
                                       preferred_element_type=_F32) + breg_ref[...]

    pred_rows = pl.pallas_call(
        mlp_body,
        grid=(n_tiles,),
        in_specs=[
            pl.BlockSpec((B, L, TN), lambda i: (0, 0, i)),
            pl.BlockSpec((RT, D), lambda i: (i, 0)),
            pl.BlockSpec((RT, D), lambda i: (i, 0)),
            pl.BlockSpec((TN, D), lambda i: (i, 0)),
            pl.BlockSpec((RT, D), lambda i: (i, 0)),
            pl.BlockSpec((L, D), lambda i: (0, 0)),
            pl.BlockSpec((n_mlp, TEMP, TEMP), lambda i: (0, 0, 0)),
            pl.BlockSpec((n_mlp, 1, TEMP), lambda i: (0, 0, 0)),
            pl.BlockSpec((n_mlp, TEMP, TEMP), lambda i: (0, 0, 0)),
            pl.BlockSpec((n_mlp, 1, TEMP), lambda i: (0, 0, 0)),
            pl.BlockSpec((TEMP, OUT), lambda i: (0, 0)),
            pl.BlockSpec((1, OUT), lambda i: (0, 0)),
        ],
        out_specs=pl.BlockSpec((RT, OUT), lambda i: (i, 0)),
        out_shape=jax.ShapeDtypeStruct((RW, OUT), _F32),
        scratch_shapes=[pltpu.VMEM((TN, B, D), _F32)],
        compiler_params=pltpu.CompilerParams(
            vmem_limit_bytes=100 * 1024 * 1024),
    )(x_bln, t_rows, d_rows, proj, h2_rows, WtsT, W1T, b1r, W2T, b2r, WregT,
      bregr)

    pred = jnp.transpose(pred_rows.reshape(N, B, OUT), (1, 2, 0))
    return pred[..., None]
